# baseline (speedup 1.0000x reference)
import functools

import jax
import jax.numpy as jnp
from jax import lax
from jax.experimental import pallas as pl
from jax.experimental.pallas import tpu as pltpu
from jax.experimental.pallas import tpu_sc as plsc

_N = 1_000_000
_D = 64
_NP = _N // 2
_T = 1600
_TP = _T // 2
_G = 80
_NG = _TP // _G
_NCHUNKS = _N // _T
_NW = 32
_L = 16


def _bucket(d):
    x = jnp.maximum(d - 1, 0)
    b = lax.bitcast_convert_type(x.astype(jnp.float32), jnp.int32)
    return jnp.clip((b >> 23) - 126, 0, 10)


def _body(dist_hbm, ptable_hbm, out_hbm, dist_v, idx_v, rows_v, sem):
    cid = lax.axis_index("c")
    sid = lax.axis_index("s")
    wid = sid * 2 + cid
    nk = (_NCHUNKS + _NW - 1 - wid) // _NW

    lanes = lax.iota(jnp.int32, _L)
    pat_e = (lanes * 2) & (_L - 1)
    pat_o = (lanes * 2 + 1) & (_L - 1)
    low = lanes < (_L // 2)

    def chunk_body(i, carry):
        k = wid + i * _NW
        pltpu.sync_copy(dist_hbm.at[pl.ds(k * _T, _T)], dist_v)

        def vec_body(q, c2):
            a = _bucket(dist_v[pl.ds(q * 2 * _L, _L)])
            b = _bucket(dist_v[pl.ds(q * 2 * _L + _L, _L)])
            ev = jnp.where(low, a.at[pat_e].get(mode="promise_in_bounds"),
                           b.at[pat_e].get(mode="promise_in_bounds"))
            od = jnp.where(low, a.at[pat_o].get(mode="promise_in_bounds"),
                           b.at[pat_o].get(mode="promise_in_bounds"))
            p = ev * 11 + od
            idx_v[q // (_G // _L), pl.ds((q % (_G // _L)) * _L, _L)] = p
            return c2

        lax.fori_loop(0, _TP // _L, vec_body, 0)

        handles = []
        for g in range(_NG):
            handles.append(
                pltpu.async_copy(
                    ptable_hbm.at[idx_v.at[g]],
                    rows_v.at[pl.ds(g * _G, _G)],
                    sem,
                )
            )
        for h in handles:
            h.wait()

        pltpu.sync_copy(rows_v, out_hbm.at[pl.ds(k * _TP, _TP)])
        return carry

    lax.fori_loop(0, nk, chunk_body, 0)


@functools.cache
def _build():
    mesh = plsc.VectorSubcoreMesh(core_axis_name="c", subcore_axis_name="s")
    return pl.kernel(
        _body,
        mesh=mesh,
        out_type=jax.ShapeDtypeStruct((_NP, 2 * _D), jnp.float32),
        scratch_types=[
            pltpu.VMEM((_T,), jnp.int32),
            pltpu.VMEM((_NG, _G), jnp.int32),
            pltpu.VMEM((_TP, 2 * _D), jnp.float32),
            pltpu.SemaphoreType.DMA,
        ],
    )


def kernel(distances, table):
    ptable = jnp.concatenate(
        [jnp.repeat(table, 11, axis=0), jnp.tile(table, (11, 1))], axis=1
    )
    out = _build()(distances, ptable)
    return out.reshape(_N, _D)


# speedup vs baseline: 15.5045x; 15.5045x over previous
import functools

import jax
import jax.numpy as jnp
from jax import lax
from jax.experimental import pallas as pl
from jax.experimental.pallas import tpu as pltpu
from jax.experimental.pallas import tpu_sc as plsc

_N = 1_000_000
_D = 64
_T = 800
_NCHUNKS = _N // _T
_NW = 32
_L = 16


def _bucket(d):
    x = jnp.maximum(d - 1, 0)
    b = lax.bitcast_convert_type(x.astype(jnp.float32), jnp.int32)
    return jnp.clip((b >> 23) - 126, 0, 10)


def _body(dist_hbm, table_hbm, out_hbm, table_v, dist_v, rows_v, sem):
    cid = lax.axis_index("c")
    sid = lax.axis_index("s")
    wid = sid * 2 + cid
    nk = (_NCHUNKS + _NW - 1 - wid) // _NW

    pltpu.sync_copy(table_hbm, table_v)

    cols = [lax.iota(jnp.int32, _L) + q * _L for q in range(_D // _L)]
    lane_consts = [jnp.full((_L,), r, jnp.int32) for r in range(_L)]

    def chunk_body(i, carry):
        k = wid + i * _NW
        pltpu.sync_copy(dist_hbm.at[pl.ds(k * _T, _T)], dist_v)

        def grp_body(j, c2):
            iv = _bucket(dist_v[pl.ds(j * _L, _L)]) * _D
            for r in range(_L):
                base = iv.at[lane_consts[r]].get(mode="promise_in_bounds")
                for q in range(_D // _L):
                    v = plsc.load_gather(table_v, [base + cols[q]])
                    rows_v[j * _L + r, pl.ds(q * _L, _L)] = v
            return c2

        lax.fori_loop(0, _T // _L, grp_body, 0)

        pltpu.sync_copy(rows_v, out_hbm.at[pl.ds(k * _T, _T)])
        return carry

    lax.fori_loop(0, nk, chunk_body, 0)


@functools.cache
def _build():
    mesh = plsc.VectorSubcoreMesh(core_axis_name="c", subcore_axis_name="s")
    return pl.kernel(
        _body,
        mesh=mesh,
        out_type=jax.ShapeDtypeStruct((_N, _D), jnp.float32),
        scratch_types=[
            pltpu.VMEM((11 * _D,), jnp.float32),
            pltpu.VMEM((_T,), jnp.int32),
            pltpu.VMEM((_T, _D), jnp.float32),
            pltpu.SemaphoreType.DMA,
        ],
        compiler_params=pltpu.CompilerParams(needs_layout_passes=False),
    )


def kernel(distances, table):
    return _build()(distances, table.reshape(-1))


# speedup vs baseline: 18.9843x; 1.2244x over previous
import functools

import jax
import jax.numpy as jnp
from jax import lax
from jax.experimental import pallas as pl
from jax.experimental.pallas import tpu as pltpu
from jax.experimental.pallas import tpu_sc as plsc

_N = 1_000_000
_D = 64
_T = 400
_NCHUNKS = _N // _T
_NW = 32
_L = 16
_NK2 = (_NCHUNKS // _NW + 2) // 2


def _bucket(d):
    x = jnp.maximum(d - 1, 0)
    b = lax.bitcast_convert_type(x.astype(jnp.float32), jnp.int32)
    return jnp.clip((b >> 23) - 126, 0, 10)


def _body(dist_hbm, table_hbm, out_hbm,
          table_v, dist_v0, dist_v1, rows_v0, rows_v1,
          sem_d0, sem_d1, sem_o0, sem_o1):
    cid = lax.axis_index("c")
    sid = lax.axis_index("s")
    wid = sid * 2 + cid

    pltpu.sync_copy(table_hbm, table_v)

    cols = [lax.iota(jnp.int32, _L) + q * _L for q in range(_D // _L)]
    lane_consts = [jnp.full((_L,), r, jnp.int32) for r in range(_L)]

    def expand(dist_v, rows_v):
        def grp_body(j, c2):
            iv = _bucket(dist_v[pl.ds(j * _L, _L)]) * _D
            for r in range(_L):
                base = iv.at[lane_consts[r]].get(mode="promise_in_bounds")
                for q in range(_D // _L):
                    v = plsc.load_gather(table_v, [base + cols[q]])
                    rows_v[j * _L + r, pl.ds(q * _L, _L)] = v
            return c2

        lax.fori_loop(0, _T // _L, grp_body, 0)

    def half(i, j, dist_v, rows_v, dist_nv, sem_d, sem_d_next, sem_o):
        k = wid + i * _NW

        @pl.when(k < _NCHUNKS)
        def _():
            kn = k + _NW

            @pl.when(kn < _NCHUNKS)
            def _():
                pltpu.async_copy(
                    dist_hbm.at[pl.ds(kn * _T, _T)], dist_nv, sem_d_next)

            pltpu.make_async_copy(
                dist_hbm.at[pl.ds(k * _T, _T)], dist_v, sem_d).wait()

            @pl.when(j > 0)
            def _():
                pltpu.make_async_copy(
                    rows_v, out_hbm.at[pl.ds(k * _T, _T)], sem_o).wait()

            expand(dist_v, rows_v)
            pltpu.async_copy(rows_v, out_hbm.at[pl.ds(k * _T, _T)], sem_o)

    pltpu.async_copy(dist_hbm.at[pl.ds(wid * _T, _T)], dist_v0, sem_d0)

    def iter_body(j, carry):
        half(2 * j, j, dist_v0, rows_v0, dist_v1, sem_d0, sem_d1, sem_o0)
        half(2 * j + 1, j, dist_v1, rows_v1, dist_v0, sem_d1, sem_d0, sem_o1)
        return carry

    lax.fori_loop(0, _NK2, iter_body, 0)

    pltpu.make_async_copy(rows_v0, out_hbm.at[pl.ds(wid * _T, _T)], sem_o0).wait()
    pltpu.make_async_copy(rows_v1, out_hbm.at[pl.ds(wid * _T, _T)], sem_o1).wait()


@functools.cache
def _build():
    mesh = plsc.VectorSubcoreMesh(core_axis_name="c", subcore_axis_name="s")
    return pl.kernel(
        _body,
        mesh=mesh,
        out_type=jax.ShapeDtypeStruct((_N, _D), jnp.float32),
        scratch_types=[
            pltpu.VMEM((11 * _D,), jnp.float32),
            pltpu.VMEM((_T,), jnp.int32),
            pltpu.VMEM((_T,), jnp.int32),
            pltpu.VMEM((_T, _D), jnp.float32),
            pltpu.VMEM((_T, _D), jnp.float32),
            pltpu.SemaphoreType.DMA,
            pltpu.SemaphoreType.DMA,
            pltpu.SemaphoreType.DMA,
            pltpu.SemaphoreType.DMA,
        ],
        compiler_params=pltpu.CompilerParams(needs_layout_passes=False),
    )


def kernel(distances, table):
    return _build()(distances, table.reshape(-1))


# speedup vs baseline: 28.0088x; 1.4754x over previous
import functools

import jax
import jax.numpy as jnp
from jax import lax
from jax.experimental import pallas as pl
from jax.experimental.pallas import tpu as pltpu
from jax.experimental.pallas import tpu_sc as plsc

_N = 1_000_000
_D = 64
_T = 400
_NCHUNKS = _N // _T
_NW = 32
_L = 16
_NK2 = (_NCHUNKS // _NW + 2) // 2


def _bucket(d):
    x = jnp.maximum(d - 1, 0)
    b = lax.bitcast_convert_type(x.astype(jnp.float32), jnp.int32)
    return jnp.clip((b >> 23) - 126, 0, 10)


def _body(dist_hbm, table_hbm, out_hbm,
          table_v, dist_v0, dist_v1, rows_v0, rows_v1,
          sem_d0, sem_d1, sem_o0, sem_o1):
    cid = lax.axis_index("c")
    sid = lax.axis_index("s")
    wid = sid * 2 + cid

    pltpu.sync_copy(table_hbm, table_v)

    cols = [lax.iota(jnp.int32, _L) + q * _L for q in range(_D // _L)]
    lane_consts = [jnp.full((_L,), r, jnp.int32) for r in range(_L)]

    def expand(dist_v, rows_v):
        @plsc.parallel_loop(0, _T // _L, 1, unroll=2)
        def grp_body(j):
            iv = _bucket(dist_v[pl.ds(j * _L, _L)]) * _D
            for r in range(_L):
                base = iv.at[lane_consts[r]].get(mode="promise_in_bounds")
                for q in range(_D // _L):
                    v = plsc.load_gather(table_v, [base + cols[q]])
                    rows_v[j * _L + r, pl.ds(q * _L, _L)] = v

    def half(i, j, dist_v, rows_v, dist_nv, sem_d, sem_d_next, sem_o):
        k = wid + i * _NW

        @pl.when(k < _NCHUNKS)
        def _():
            kn = k + _NW

            @pl.when(kn < _NCHUNKS)
            def _():
                pltpu.async_copy(
                    dist_hbm.at[pl.ds(kn * _T, _T)], dist_nv, sem_d_next)

            pltpu.make_async_copy(
                dist_hbm.at[pl.ds(k * _T, _T)], dist_v, sem_d).wait()

            @pl.when(j > 0)
            def _():
                pltpu.make_async_copy(
                    rows_v, out_hbm.at[pl.ds(k * _T, _T)], sem_o).wait()

            expand(dist_v, rows_v)
            pltpu.async_copy(rows_v, out_hbm.at[pl.ds(k * _T, _T)], sem_o)

    pltpu.async_copy(dist_hbm.at[pl.ds(wid * _T, _T)], dist_v0, sem_d0)

    def iter_body(j, carry):
        half(2 * j, j, dist_v0, rows_v0, dist_v1, sem_d0, sem_d1, sem_o0)
        half(2 * j + 1, j, dist_v1, rows_v1, dist_v0, sem_d1, sem_d0, sem_o1)
        return carry

    lax.fori_loop(0, _NK2, iter_body, 0)

    pltpu.make_async_copy(rows_v0, out_hbm.at[pl.ds(wid * _T, _T)], sem_o0).wait()
    pltpu.make_async_copy(rows_v1, out_hbm.at[pl.ds(wid * _T, _T)], sem_o1).wait()


@functools.cache
def _build():
    mesh = plsc.VectorSubcoreMesh(core_axis_name="c", subcore_axis_name="s")
    return pl.kernel(
        _body,
        mesh=mesh,
        out_type=jax.ShapeDtypeStruct((_N, _D), jnp.float32),
        scratch_types=[
            pltpu.VMEM((11 * _D,), jnp.float32),
            pltpu.VMEM((_T,), jnp.int32),
            pltpu.VMEM((_T,), jnp.int32),
            pltpu.VMEM((_T, _D), jnp.float32),
            pltpu.VMEM((_T, _D), jnp.float32),
            pltpu.SemaphoreType.DMA,
            pltpu.SemaphoreType.DMA,
            pltpu.SemaphoreType.DMA,
            pltpu.SemaphoreType.DMA,
        ],
        compiler_params=pltpu.CompilerParams(needs_layout_passes=False),
    )


def kernel(distances, table):
    return _build()(distances, table.reshape(-1))


# speedup vs baseline: 28.0540x; 1.0016x over previous
import functools

import jax
import jax.numpy as jnp
from jax import lax
from jax.experimental import pallas as pl
from jax.experimental.pallas import tpu as pltpu
from jax.experimental.pallas import tpu_sc as plsc

_N = 1_000_000
_D = 64
_T = 400
_NCHUNKS = _N // _T
_NW = 32
_L = 16
_NK2 = (_NCHUNKS // _NW + 2) // 2


def _bucket(d):
    x = jnp.maximum(d - 1, 0)
    b = lax.bitcast_convert_type(x.astype(jnp.float32), jnp.int32)
    return jnp.clip((b >> 23) - 126, 0, 10)


def _body(dist_hbm, table_hbm, out_hbm,
          table_v, dist_v0, dist_v1, rows_v0, rows_v1,
          sem_d0, sem_d1, sem_o0, sem_o1):
    cid = lax.axis_index("c")
    sid = lax.axis_index("s")
    wid = sid * 2 + cid

    pltpu.sync_copy(table_hbm, table_v)

    cols = [lax.iota(jnp.int32, _L) + q * _L for q in range(_D // _L)]
    lane_consts = [jnp.full((_L,), r, jnp.int32) for r in range(_L)]

    def expand(dist_v, rows_v):
        @plsc.parallel_loop(0, _T // _L, 1, unroll=4)
        def grp_body(j):
            iv = _bucket(dist_v[pl.ds(j * _L, _L)]) * _D
            for r in range(_L):
                base = iv.at[lane_consts[r]].get(mode="promise_in_bounds")
                for q in range(_D // _L):
                    v = plsc.load_gather(table_v, [base + cols[q]])
                    rows_v[j * _L + r, pl.ds(q * _L, _L)] = v

    def half(i, j, dist_v, rows_v, dist_nv, sem_d, sem_d_next, sem_o):
        k = wid + i * _NW

        @pl.when(k < _NCHUNKS)
        def _():
            kn = k + _NW

            @pl.when(kn < _NCHUNKS)
            def _():
                pltpu.async_copy(
                    dist_hbm.at[pl.ds(kn * _T, _T)], dist_nv, sem_d_next)

            pltpu.make_async_copy(
                dist_hbm.at[pl.ds(k * _T, _T)], dist_v, sem_d).wait()

            @pl.when(j > 0)
            def _():
                pltpu.make_async_copy(
                    rows_v, out_hbm.at[pl.ds(k * _T, _T)], sem_o).wait()

            expand(dist_v, rows_v)
            pltpu.async_copy(rows_v, out_hbm.at[pl.ds(k * _T, _T)], sem_o)

    pltpu.async_copy(dist_hbm.at[pl.ds(wid * _T, _T)], dist_v0, sem_d0)

    def iter_body(j, carry):
        half(2 * j, j, dist_v0, rows_v0, dist_v1, sem_d0, sem_d1, sem_o0)
        half(2 * j + 1, j, dist_v1, rows_v1, dist_v0, sem_d1, sem_d0, sem_o1)
        return carry

    lax.fori_loop(0, _NK2, iter_body, 0)

    pltpu.make_async_copy(rows_v0, out_hbm.at[pl.ds(wid * _T, _T)], sem_o0).wait()
    pltpu.make_async_copy(rows_v1, out_hbm.at[pl.ds(wid * _T, _T)], sem_o1).wait()


@functools.cache
def _build():
    mesh = plsc.VectorSubcoreMesh(core_axis_name="c", subcore_axis_name="s")
    return pl.kernel(
        _body,
        mesh=mesh,
        out_type=jax.ShapeDtypeStruct((_N, _D), jnp.float32),
        scratch_types=[
            pltpu.VMEM((11 * _D,), jnp.float32),
            pltpu.VMEM((_T,), jnp.int32),
            pltpu.VMEM((_T,), jnp.int32),
            pltpu.VMEM((_T, _D), jnp.float32),
            pltpu.VMEM((_T, _D), jnp.float32),
            pltpu.SemaphoreType.DMA,
            pltpu.SemaphoreType.DMA,
            pltpu.SemaphoreType.DMA,
            pltpu.SemaphoreType.DMA,
        ],
        compiler_params=pltpu.CompilerParams(needs_layout_passes=False),
    )


def kernel(distances, table):
    return _build()(distances, table.reshape(-1))
